# Initial kernel scaffold; baseline (speedup 1.0000x reference)
import jax
import jax.numpy as jnp
from jax.experimental import pallas as pl


def kernel(x, adj, weight, bias):
    raise NotImplementedError("write your pallas kernel here")


# speedup vs baseline: 1.5214x; 1.5214x over previous
import jax
import jax.numpy as jnp
from jax.experimental import pallas as pl
from jax.experimental.pallas import tpu as pltpu


def _round_up(x, m):
    return ((x + m - 1) // m) * m


def _support_body(x_ref, w_ref, o_ref):
    x = x_ref[...].astype(jnp.bfloat16)
    w = w_ref[...].astype(jnp.bfloat16)
    o_ref[...] = jnp.dot(
        x, w, preferred_element_type=jnp.float32
    ).astype(o_ref.dtype)


def _aggregate_body(adj_ref, s_ref, b_ref, o_ref):
    adj = adj_ref[...].astype(jnp.bfloat16)
    acc = jnp.dot(adj, s_ref[...], preferred_element_type=jnp.float32)
    o_ref[...] = acc + b_ref[...]


def kernel(x, adj, weight, bias):
    n, f_in = x.shape
    f_out = weight.shape[1]

    f_in_p = _round_up(f_in, 128)
    f_out_p = _round_up(f_out, 128)

    tm = 512
    n_p = _round_up(n, tm)

    x = x.astype(jnp.float32)
    if (n_p, f_in_p) != (n, f_in):
        x = jnp.pad(x, ((0, n_p - n), (0, f_in_p - f_in)))
    w = weight.astype(jnp.float32)
    if (f_in_p, f_out_p) != (f_in, f_out):
        w = jnp.pad(w, ((0, f_in_p - f_in), (0, f_out_p - f_out)))
    adj_p = adj if n_p == n else jnp.pad(adj, ((0, n_p - n), (0, n_p - n)))
    if bias is None:
        b = jnp.zeros((1, f_out_p), jnp.float32)
    else:
        b = jnp.pad(bias.reshape(1, f_out).astype(jnp.float32),
                    ((0, 0), (0, f_out_p - f_out)))

    tms = min(2048, n_p)
    support = pl.pallas_call(
        _support_body,
        out_shape=jax.ShapeDtypeStruct((n_p, f_out_p), jnp.bfloat16),
        grid=(n_p // tms,),
        in_specs=[
            pl.BlockSpec((tms, f_in_p), lambda i: (i, 0)),
            pl.BlockSpec((f_in_p, f_out_p), lambda i: (0, 0)),
        ],
        out_specs=pl.BlockSpec((tms, f_out_p), lambda i: (i, 0)),
        compiler_params=pltpu.CompilerParams(
            dimension_semantics=("parallel",),
            vmem_limit_bytes=32 << 20,
        ),
    )(x, w)

    out_p = pl.pallas_call(
        _aggregate_body,
        out_shape=jax.ShapeDtypeStruct((n_p, f_out_p), jnp.float32),
        grid=(n_p // tm,),
        in_specs=[
            pl.BlockSpec((tm, n_p), lambda i: (i, 0)),
            pl.BlockSpec((n_p, f_out_p), lambda i: (0, 0)),
            pl.BlockSpec((1, f_out_p), lambda i: (0, 0)),
        ],
        out_specs=pl.BlockSpec((tm, f_out_p), lambda i: (i, 0)),
        compiler_params=pltpu.CompilerParams(
            dimension_semantics=("parallel",),
            vmem_limit_bytes=48 << 20,
        ),
    )(adj_p, support, b)

    return out_p[:n, :f_out]
